# baseline (speedup 1.0000x reference)
import jax
import jax.numpy as jnp
from jax.experimental import pallas as pl
from jax.experimental.pallas import tpu as pltpu


def _gru_forward_final(x, W_ih, W_hh, b):
    Hh = W_hh.shape[0]

    def step(h, xt):
        gi = xt @ W_ih
        gh = h @ W_hh
        z = jax.nn.sigmoid(gi[:, :Hh] + gh[:, :Hh] + b[:Hh])
        r = jax.nn.sigmoid(gi[:, Hh:2 * Hh] + gh[:, Hh:2 * Hh] + b[Hh:2 * Hh])
        n = jnp.tanh(gi[:, 2 * Hh:] + r * gh[:, 2 * Hh:] + b[2 * Hh:])
        h_new = (1.0 - z) * h + z * n
        return h_new, None

    h0 = jnp.zeros((x.shape[0], Hh), x.dtype)
    xs = jnp.transpose(x, (1, 0, 2))
    final, _ = jax.lax.scan(step, h0, xs)
    return final


def _attention_gcn(h_nodes, edge_index, W, a, n_nodes):
    loop = jnp.arange(n_nodes, dtype=edge_index.dtype)
    src = jnp.concatenate([edge_index[0], loop])
    dst = jnp.concatenate([edge_index[1], loop])
    hw = h_nodes @ W
    Hd = hw.shape[1]
    a_src = hw @ a[:Hd]
    a_dst = hw @ a[Hd:]
    e = jax.nn.leaky_relu(a_src[src] + a_dst[dst], 0.2)
    m = jax.ops.segment_max(e, dst, num_segments=n_nodes)
    ex = jnp.exp(e - m[dst])
    den = jax.ops.segment_sum(ex, dst, num_segments=n_nodes)
    w = ex / (den[dst] + 1e-16)
    out = jax.ops.segment_sum(w[:, None] * hw[src], dst, num_segments=n_nodes)
    return jax.nn.elu(out)


def _head_kernel(d_ref, s_ref, Wd_ref, Ws_ref, b_ref, P1_ref, pb1_ref,
                 P2_ref, pb2_ref, out_ref):
    f = jnp.maximum(
        d_ref[...] @ Wd_ref[...] + s_ref[...] @ Ws_ref[...] + b_ref[...][None, :],
        0.0)
    h1 = jnp.maximum(f @ P1_ref[...] + pb1_ref[...][None, :], 0.0)
    logits = h1 @ P2_ref[...] + pb2_ref[...][None, :]
    out_ref[...] = jax.nn.sigmoid(logits)


def kernel(x, edge_index1, edge_index2, W_ih, W_hh, b_gru, W1, a1, W2, a2,
           Wd, Ws, b_rff, P1, pb1, P2, pb2):
    Bc, Nc, Tc, Dc = x.shape
    xf = x.reshape(Bc * Nc, Tc, Dc)
    final = _gru_forward_final(xf, W_ih, W_hh, b_gru)
    s1 = _attention_gcn(final, edge_index1, W1, a1, Bc * Nc)
    s2 = _attention_gcn(final, edge_index2, W2, a2, Bc * Nc)
    s = (s1 + s2) / 2.0

    out = pl.pallas_call(
        _head_kernel,
        out_shape=jax.ShapeDtypeStruct((Bc * Nc, 1), jnp.float32),
    )(final, s, Wd, Ws, b_rff, P1, pb1, P2, pb2)
    return out.reshape(Bc, Nc)


# speedup vs baseline: 6.2288x; 6.2288x over previous
import functools

import jax
import jax.numpy as jnp
from jax import lax
from jax.experimental import pallas as pl
from jax.experimental.pallas import tpu as pltpu
from jax.experimental.pallas import tpu_sc as plsc

N = 10000
T = 12
D = 128
H = 128
E = 320000
NP = 10240
NTILES = 16
CHUNK = 128
NCHUNKS = 157
EPAD = NTILES * NCHUNKS * CHUNK
ROWS_PER_TILE = NP // NTILES
NPH = 3584
NPASS = 3
NACC = NPH + 8
ACC_PER_TILE = NPH // NTILES
NEG = -1e30


def _gru_body(x_ref, wih_ref, whh_ref, b_ref, out_ref):
    blk = x_ref.shape[0]
    wih = wih_ref[...]
    whh = whh_ref[...]
    b = b_ref[...]
    h = jnp.zeros((blk, H), jnp.float32)
    for t in range(T):
        xt = x_ref[:, t * D:(t + 1) * D]
        gi = jnp.dot(xt, wih, preferred_element_type=jnp.float32)
        gh = jnp.dot(h, whh, preferred_element_type=jnp.float32)
        z = jax.nn.sigmoid(gi[:, :H] + gh[:, :H] + b[:, :H])
        r = jax.nn.sigmoid(gi[:, H:2 * H] + gh[:, H:2 * H] + b[:, H:2 * H])
        n = jnp.tanh(gi[:, 2 * H:] + r * gh[:, 2 * H:] + b[:, 2 * H:])
        h = (1.0 - z) * h + z * n
    out_ref[...] = h


def _gru_final(x2d, W_ih, W_hh, b_gru):
    blk = 1000
    return pl.pallas_call(
        _gru_body,
        grid=(N // blk,),
        in_specs=[
            pl.BlockSpec((blk, T * D), lambda i: (i, 0)),
            pl.BlockSpec((D, 3 * H), lambda i: (0, 0)),
            pl.BlockSpec((H, 3 * H), lambda i: (0, 0)),
            pl.BlockSpec((1, 3 * H), lambda i: (0, 0)),
        ],
        out_specs=pl.BlockSpec((blk, H), lambda i: (i, 0)),
        out_shape=jax.ShapeDtypeStruct((N, H), jnp.float32),
    )(x2d, W_ih, W_hh, b_gru.reshape(1, 3 * H))


def _rowdot(u, v):
    return lax.dot_general(u, v, (((1,), (1,)), ((), ())),
                           preferred_element_type=jnp.float32)


def _prep_body(final_ref, w1_ref, w2_ref, a1s_ref, a1d_ref, a2s_ref, a2d_ref,
               asd_ref, wself_ref):
    final = final_ref[...]
    rows = []
    ms = []
    for (w_ref, as_ref, ad_ref) in ((w1_ref, a1s_ref, a1d_ref),
                                    (w2_ref, a2s_ref, a2d_ref)):
        cs = _rowdot(as_ref[...], w_ref[...])
        cd = _rowdot(ad_ref[...], w_ref[...])
        asrc = _rowdot(cs, final)
        adst = _rowdot(cd, final)
        m = jnp.max(asrc) + jnp.max(adst)
        m = jnp.maximum(m, 0.2 * m)
        rows.append((asrc, adst))
        ms.append(m)

    asd_ref[...] = jnp.full((6, NP), NEG, jnp.float32)
    asd_ref[0:4, 0:N] = jnp.concatenate(
        [rows[0][0], rows[0][1], rows[1][0], rows[1][1]], axis=0)
    asd_ref[4:5, :] = jnp.full((1, NP), ms[0], jnp.float32)
    asd_ref[5:6, :] = jnp.full((1, NP), ms[1], jnp.float32)

    for layer in range(2):
        u = rows[layer][0] + rows[layer][1]
        e = jnp.maximum(u, 0.2 * u)
        wself_ref[layer:layer + 1, :] = jnp.exp(e - ms[layer])


def _prep(final, W1, a1, W2, a2):
    return pl.pallas_call(
        _prep_body,
        out_shape=(
            jax.ShapeDtypeStruct((6, NP), jnp.float32),
            jax.ShapeDtypeStruct((2, N), jnp.float32),
        ),
    )(final, W1, W2,
      a1[:H].reshape(1, H), a1[H:].reshape(1, H),
      a2[:H].reshape(1, H), a2[H:].reshape(1, H))


def _splat(v16, r):
    lanes = lax.iota(jnp.int32, 16)
    tot = jnp.sum(jnp.where(lanes == r, v16, 0.0))
    return jnp.broadcast_to(tot, (16,))


def _edge_body(final_hbm, asd_hbm, src_hbm, dst_hbm, num_out, den_out,
               den_slots, as_v, ad_v, mv, srcb, dstb, dstadj, den_l, rows,
               zbuf, dstage, rbuf, num_acc, sem):
    s = lax.axis_index("s")

    z16 = jnp.zeros((16,), jnp.float32)
    for i in range(16):
        for j in range(8):
            zbuf[i, pl.ds(j * 16, 16)] = z16

    lanes = lax.iota(jnp.int32, 16)
    seg = NP // NTILES

    def layer(l, _):
        pltpu.sync_copy(asd_hbm.at[2 * l], as_v)
        pltpu.sync_copy(asd_hbm.at[2 * l + 1], ad_v)
        pltpu.sync_copy(asd_hbm.at[4 + l, pl.ds(0, 16)], mv)
        pltpu.sync_copy(src_hbm.at[l, s], srcb)
        pltpu.sync_copy(dst_hbm.at[l, s], dstb)

        def zero_den(g, _):
            plsc.store_scatter(den_l, [g * 16 + lanes], z16)
            return 0
        lax.fori_loop(0, NP // 16, zero_den, 0)

        mvec = mv[...]

        def half(hp, _):
            base = hp * NPH

            def zero_num(i, _):
                pltpu.sync_copy(
                    zbuf, num_acc.at[pl.ds(s * ACC_PER_TILE + i * 16, 16)])
                return 0
            lax.fori_loop(0, ACC_PER_TILE // 16, zero_num, 0)

            @pl.when(s == 0)
            def _():
                pltpu.sync_copy(zbuf, num_acc.at[pl.ds(NPH - 8, 16)])

            plsc.subcore_barrier()

            def chunk(g, _):
                pltpu.async_copy(final_hbm.at[srcb.at[g]], rows, sem).wait()
                for j in range(8):
                    sv = srcb[g, pl.ds(j * 16, 16)]
                    dv = dstb[g, pl.ds(j * 16, 16)]
                    a_s = plsc.load_gather(as_v, [sv])
                    a_d = plsc.load_gather(ad_v, [dv])
                    u = a_s + a_d
                    e = jnp.maximum(u, 0.2 * u)
                    w16 = jnp.exp(e - mvec)

                    @pl.when(hp == 0)
                    def _():
                        plsc.addupdate_scatter(den_l, [dv], w16)

                    dva = dv - base
                    ok = (dva >= 0) & (dva < NPH)
                    dva = jnp.where(ok, dva, NPH)
                    dstadj[0, pl.ds(j * 16, 16)] = dva
                    for r in range(16):
                        ws = _splat(w16, r)
                        row = j * 16 + r
                        for q in range(8):
                            sl = pl.ds(q * 16, 16)
                            rows[row, sl] = rows[row, sl] * ws
                pltpu.sync_copy(rows, num_acc.at[dstadj.at[0]], add=True)
                return 0

            lax.fori_loop(0, NCHUNKS, chunk, 0)
            plsc.subcore_barrier()
            pltpu.sync_copy(
                num_acc.at[pl.ds(s * ACC_PER_TILE, ACC_PER_TILE)],
                num_out.at[l, pl.ds(base + s * ACC_PER_TILE, ACC_PER_TILE)])
            plsc.subcore_barrier()
            return 0

        lax.fori_loop(0, NPASS, half, 0)

        pltpu.sync_copy(den_l, den_slots.at[s])
        plsc.subcore_barrier()
        for t in range(NTILES):
            pltpu.sync_copy(den_slots.at[t, pl.ds(s * seg, seg)],
                            dstage.at[t])

        def red(k, _):
            sl = pl.ds(k * 16, 16)
            acc = dstage[0, sl]
            for t in range(1, NTILES):
                acc = acc + dstage[t, sl]
            rbuf[sl] = acc
            return 0
        lax.fori_loop(0, seg // 16, red, 0)
        pltpu.sync_copy(rbuf, den_out.at[l, pl.ds(s * seg, seg)])
        plsc.subcore_barrier()
        return 0

    lax.fori_loop(0, 2, layer, 0)


def _edges(final_pad, asd, src_all, dst_all):
    mesh = plsc.VectorSubcoreMesh(
        core_axis_name="c", subcore_axis_name="s", num_cores=1)
    f = pl.kernel(
        _edge_body,
        out_type=(
            jax.ShapeDtypeStruct((2, NPASS * NPH, H), jnp.float32),
            jax.ShapeDtypeStruct((2, NP), jnp.float32),
            jax.ShapeDtypeStruct((NTILES, NP), jnp.float32),
        ),
        mesh=mesh,
        scratch_types=[
            pltpu.VMEM((NP,), jnp.float32),
            pltpu.VMEM((NP,), jnp.float32),
            pltpu.VMEM((16,), jnp.float32),
            pltpu.VMEM((NCHUNKS, CHUNK), jnp.int32),
            pltpu.VMEM((NCHUNKS, CHUNK), jnp.int32),
            pltpu.VMEM((1, CHUNK), jnp.int32),
            pltpu.VMEM((NP,), jnp.float32),
            pltpu.VMEM((CHUNK, H), jnp.float32),
            pltpu.VMEM((16, 128), jnp.float32),
            pltpu.VMEM((NTILES, NP // NTILES), jnp.float32),
            pltpu.VMEM((NP // NTILES,), jnp.float32),
            pltpu.VMEM_SHARED((NACC, H), jnp.float32),
            pltpu.SemaphoreType.DMA,
        ],
        compiler_params=pltpu.CompilerParams(needs_layout_passes=False),
    )
    return f(final_pad, asd, src_all, dst_all)


def _elu(x):
    return jnp.where(x > 0, x, jnp.exp(jnp.minimum(x, 0.0)) - 1.0)


def _fusion_body(final_ref, num1_ref, num2_ref, den1_ref, den2_ref,
                 ws1_ref, ws2_ref, w1_ref, w2_ref, wd_ref, wsm_ref, b_ref,
                 p1_ref, pb1_ref, p2_ref, pb2_ref, out_ref):
    final = final_ref[...]
    s_acc = None
    for (num_ref, den_ref, ws_ref, w_ref) in (
            (num1_ref, den1_ref, ws1_ref, w1_ref),
            (num2_ref, den2_ref, ws2_ref, w2_ref)):
        wself = ws_ref[...]
        num = num_ref[...] + wself * final
        den = den_ref[...] + wself + 1e-30
        agg = num / den
        sl = _elu(jnp.dot(agg, w_ref[...], preferred_element_type=jnp.float32))
        s_acc = sl if s_acc is None else s_acc + sl
    s = s_acc * 0.5
    f = jnp.maximum(
        jnp.dot(final, wd_ref[...], preferred_element_type=jnp.float32)
        + jnp.dot(s, wsm_ref[...], preferred_element_type=jnp.float32)
        + b_ref[...], 0.0)
    h1 = jnp.maximum(
        jnp.dot(f, p1_ref[...], preferred_element_type=jnp.float32)
        + pb1_ref[...], 0.0)
    logits = jnp.dot(h1, p2_ref[...], preferred_element_type=jnp.float32) \
        + pb2_ref[...]
    out_ref[...] = jax.nn.sigmoid(logits)


def _fusion(final, num1, num2, den1, den2, ws1, ws2, W1, W2, Wd, Ws, b_rff,
            P1, pb1, P2, pb2):
    return pl.pallas_call(
        _fusion_body,
        out_shape=jax.ShapeDtypeStruct((N, 1), jnp.float32),
    )(final, num1, num2, den1, den2, ws1, ws2, W1, W2, Wd, Ws,
      b_rff.reshape(1, H), P1, pb1.reshape(1, H), P2, pb2.reshape(1, 1))


def kernel(x, edge_index1, edge_index2, W_ih, W_hh, b_gru, W1, a1, W2, a2,
           Wd, Ws, b_rff, P1, pb1, P2, pb2):
    Bc, Nc, Tc, Dc = x.shape
    x2d = x.reshape(Nc, Tc * Dc)

    final = _gru_final(x2d, W_ih, W_hh, b_gru)
    asd, wself = _prep(final, W1, a1, W2, a2)

    pad_idx = jnp.full((EPAD - E,), N, jnp.int32)
    src_all = jnp.stack([
        jnp.concatenate([edge_index1[0], pad_idx]),
        jnp.concatenate([edge_index2[0], pad_idx]),
    ]).reshape(2, NTILES, NCHUNKS, CHUNK)
    dst_all = jnp.stack([
        jnp.concatenate([edge_index1[1], pad_idx]),
        jnp.concatenate([edge_index2[1], pad_idx]),
    ]).reshape(2, NTILES, NCHUNKS, CHUNK)

    final_pad = jnp.concatenate(
        [final, jnp.zeros((NP - N, H), jnp.float32)], axis=0)

    num, den, _ = _edges(final_pad, asd, src_all, dst_all)

    num1 = num[0, :N]
    num2 = num[1, :N]
    den1 = den[0, :N].reshape(N, 1)
    den2 = den[1, :N].reshape(N, 1)
    ws1 = wself[0].reshape(N, 1)
    ws2 = wself[1].reshape(N, 1)

    out = _fusion(final, num1, num2, den1, den2, ws1, ws2,
                  W1, W2, Wd, Ws, b_rff, P1, pb1, P2, pb2)
    return out.reshape(Bc, Nc)
